# Initial kernel scaffold; baseline (speedup 1.0000x reference)
import jax
import jax.numpy as jnp
from jax.experimental import pallas as pl


def kernel(x, edge_index, batch, Wn1, Ws1, b1, Wn2, Ws2, b2, Wn3, Ws3, b3, gate_W, gate_b, reg_W, reg_b):
    raise NotImplementedError("write your pallas kernel here")


# speedup vs baseline: 4.4193x; 4.4193x over previous
import functools

import jax
import jax.numpy as jnp
from jax import lax
from jax.experimental import pallas as pl
from jax.experimental.pallas import tpu as pltpu
from jax.experimental.pallas import tpu_sc as plsc

N_NODES = 10000
FEAT = 128
G = 16
OUT = 3

NC = 2
NS = 16
NW = NC * NS
CHUNK = 128

ACC_ROWS = 10240
ROWS_PER_TILE = ACC_ROWS // NS

_mesh = plsc.VectorSubcoreMesh(core_axis_name="c", subcore_axis_name="s")


def _make_agg(cpt):
    @functools.partial(
        pl.kernel,
        out_type=jax.ShapeDtypeStruct((NC * ACC_ROWS, FEAT), jnp.float32),
        mesh=_mesh,
        scratch_types=[
            pltpu.VMEM((cpt, CHUNK), jnp.int32),
            pltpu.VMEM((cpt, CHUNK), jnp.int32),
            pltpu.VMEM((CHUNK, FEAT), jnp.float32),
            pltpu.VMEM_SHARED((ACC_ROWS, FEAT), jnp.float32),
            pltpu.SemaphoreType.DMA,
        ],
    )
    def agg(h_hbm, src_hbm, dst_hbm, zeros_hbm, out_hbm, src_v, dst_v, rows_v, acc_sh, gsem):
        cid = lax.axis_index("c")
        sid = lax.axis_index("s")
        wid = sid * NC + cid
        r0 = sid * ROWS_PER_TILE
        pltpu.sync_copy(zeros_hbm.at[pl.ds(r0, ROWS_PER_TILE)],
                        acc_sh.at[pl.ds(r0, ROWS_PER_TILE)])
        pltpu.sync_copy(src_hbm.at[wid], src_v)
        pltpu.sync_copy(dst_hbm.at[wid], dst_v)
        plsc.subcore_barrier()

        def step(j, carry):
            pltpu.async_copy(h_hbm.at[src_v.at[j]], rows_v, gsem).wait()
            pltpu.sync_copy(rows_v, acc_sh.at[dst_v.at[j]], add=True)
            return carry

        lax.fori_loop(0, cpt, step, 0)
        plsc.subcore_barrier()
        pltpu.sync_copy(acc_sh.at[pl.ds(r0, ROWS_PER_TILE)],
                        out_hbm.at[pl.ds(cid * ACC_ROWS + r0, ROWS_PER_TILE)])

    return agg


def _make_deg(ept):
    @functools.partial(
        pl.kernel,
        out_type=jax.ShapeDtypeStruct((NW, ACC_ROWS), jnp.float32),
        mesh=_mesh,
        scratch_types=[
            pltpu.VMEM((ept,), jnp.int32),
            pltpu.VMEM((ACC_ROWS,), jnp.float32),
        ],
        compiler_params=pltpu.CompilerParams(needs_layout_passes=False),
    )
    def deg(dst_hbm, out_hbm, dst_v, deg_v):
        cid = lax.axis_index("c")
        sid = lax.axis_index("s")
        wid = sid * NC + cid
        pltpu.sync_copy(dst_hbm.at[wid], dst_v)

        def zstep(i, carry):
            deg_v[pl.ds(pl.multiple_of(i * 16, 16), 16)] = jnp.zeros((16,), jnp.float32)
            return carry

        lax.fori_loop(0, ACC_ROWS // 16, zstep, 0)
        ones = jnp.ones((16,), jnp.float32)

        def estep(j, carry):
            idx = dst_v[pl.ds(pl.multiple_of(j * 16, 16), 16)]
            plsc.addupdate_scatter(deg_v, [idx], ones)
            return carry

        lax.fori_loop(0, ept // 16, estep, 0)
        pltpu.sync_copy(deg_v, out_hbm.at[wid])

    return deg


def _dense_body(relu, agg_a, agg_b, degt, h, wn, ws, b, o_ref):
    deg = jnp.maximum(jnp.sum(degt[...], axis=1, keepdims=True), 1.0)
    agg = (agg_a[...] + agg_b[...]) / deg
    y = (jnp.dot(agg, wn[...], preferred_element_type=jnp.float32)
         + jnp.dot(h[...], ws[...], preferred_element_type=jnp.float32)
         + b[...])
    o_ref[...] = jnp.maximum(y, 0.0) if relu else y


def _make_dense(relu):
    bn = 1000
    grid = N_NODES // bn
    return pl.pallas_call(
        functools.partial(_dense_body, relu),
        grid=(grid,),
        in_specs=[
            pl.BlockSpec((bn, FEAT), lambda i: (i, 0)),
            pl.BlockSpec((bn, FEAT), lambda i: (i, 0)),
            pl.BlockSpec((bn, NW), lambda i: (i, 0)),
            pl.BlockSpec((bn, FEAT), lambda i: (i, 0)),
            pl.BlockSpec((FEAT, FEAT), lambda i: (0, 0)),
            pl.BlockSpec((FEAT, FEAT), lambda i: (0, 0)),
            pl.BlockSpec((1, FEAT), lambda i: (0, 0)),
        ],
        out_specs=pl.BlockSpec((bn, FEAT), lambda i: (i, 0)),
        out_shape=jax.ShapeDtypeStruct((N_NODES, FEAT), jnp.float32),
    )


def _pool_body(agg_a, agg_b, degt, h2, wn, ws, b, gw, gb, rw, rb, bids, o_ref):
    deg = jnp.maximum(jnp.sum(degt[...], axis=1, keepdims=True), 1.0)
    agg = (agg_a[...] + agg_b[...]) / deg
    h3 = (jnp.dot(agg, wn[...], preferred_element_type=jnp.float32)
          + jnp.dot(h2[...], ws[...], preferred_element_type=jnp.float32)
          + b[...])
    gate = jnp.dot(h3, gw[...], preferred_element_type=jnp.float32) + gb[...]
    gids = lax.broadcasted_iota(jnp.int32, (N_NODES, G), 1)
    mask = bids[...] == gids
    maskf = mask.astype(jnp.float32)
    gmax = jnp.max(jnp.where(mask, gate, -1e30), axis=0, keepdims=True)
    gmax_n = jnp.sum(jnp.where(mask, gmax, 0.0), axis=1, keepdims=True)
    e = jnp.exp(gate - gmax_n)
    denom = jnp.sum(e * maskf, axis=0, keepdims=True)
    denom_n = jnp.sum(jnp.where(mask, denom, 0.0), axis=1, keepdims=True)
    wgt = e / denom_n
    wm = maskf * wgt
    pooled = lax.dot_general(wm, h3, (((0,), (0,)), ((), ())),
                             preferred_element_type=jnp.float32)
    o_ref[...] = jnp.tanh(
        jnp.dot(pooled, rw[...], preferred_element_type=jnp.float32) + rb[...])


_pool = pl.pallas_call(
    _pool_body,
    out_shape=jax.ShapeDtypeStruct((G, OUT), jnp.float32),
)


def kernel(x, edge_index, batch, Wn1, Ws1, b1, Wn2, Ws2, b2, Wn3, Ws3, b3,
           gate_W, gate_b, reg_W, reg_b):
    e = edge_index.shape[1]
    ept = -(-e // NW)
    ept = -(-ept // CHUNK) * CHUNK
    cpt = ept // CHUNK
    pad = NW * ept - e

    src = edge_index[0]
    dst = edge_index[1]
    src_p = jnp.pad(src, (0, pad)).reshape(NW, cpt, CHUNK)
    dst_p = jnp.pad(dst, (0, pad), constant_values=N_NODES).reshape(NW, cpt, CHUNK)
    dst_flat = dst_p.reshape(NW, cpt * CHUNK)
    zeros = jnp.zeros((ACC_ROWS, FEAT), jnp.float32)

    agg_call = _make_agg(cpt)
    deg_call = _make_deg(ept)

    degp = deg_call(dst_flat)
    degt = degp.T[:N_NODES]

    def split(slab):
        a = lax.slice(slab, (0, 0), (N_NODES, FEAT))
        b = lax.slice(slab, (ACC_ROWS, 0), (ACC_ROWS + N_NODES, FEAT))
        return a, b

    b1r = b1.reshape(1, FEAT)
    b2r = b2.reshape(1, FEAT)
    b3r = b3.reshape(1, FEAT)

    a1a, a1b = split(agg_call(x, src_p, dst_p, zeros))
    h1 = _make_dense(True)(a1a, a1b, degt, x, Wn1, Ws1, b1r)
    a2a, a2b = split(agg_call(h1, src_p, dst_p, zeros))
    h2 = _make_dense(True)(a2a, a2b, degt, h1, Wn2, Ws2, b2r)
    a3a, a3b = split(agg_call(h2, src_p, dst_p, zeros))
    out = _pool(a3a, a3b, degt, h2, Wn3, Ws3, b3r,
                gate_W, gate_b.reshape(1, 1), reg_W, reg_b.reshape(1, OUT),
                batch.reshape(N_NODES, 1))
    return out


# speedup vs baseline: 6.2857x; 1.4223x over previous
import functools

import jax
import jax.numpy as jnp
from jax import lax
from jax.experimental import pallas as pl
from jax.experimental.pallas import tpu as pltpu
from jax.experimental.pallas import tpu_sc as plsc

N_NODES = 10000
FEAT = 128
G = 16
OUT = 3

NC = 2
NS = 16
NW = NC * NS
CHUNK = 128

ACC_ROWS = 10240
ROWS_PER_TILE = ACC_ROWS // NS

_mesh = plsc.VectorSubcoreMesh(core_axis_name="c", subcore_axis_name="s")

SPLIT0 = 0.65


def _make_agg(cpt):
    cpt0, cpt1 = cpt
    cptm = max(cpt0, cpt1)

    @functools.partial(
        pl.kernel,
        out_type=jax.ShapeDtypeStruct((NC * ACC_ROWS, FEAT), jnp.float32),
        mesh=_mesh,
        scratch_types=[
            pltpu.VMEM((cptm, CHUNK), jnp.int32),
            pltpu.VMEM((cptm, CHUNK), jnp.int32),
            pltpu.VMEM((CHUNK, FEAT), jnp.float32),
            pltpu.VMEM_SHARED((ACC_ROWS, FEAT), jnp.float32),
            pltpu.SemaphoreType.DMA,
        ],
    )
    def agg(h_hbm, src_hbm, dst_hbm, zeros_hbm, out_hbm, src_v, dst_v, rows_v, acc_sh, gsem):
        cid = lax.axis_index("c")
        sid = lax.axis_index("s")
        wid = sid * NC + cid
        r0 = sid * ROWS_PER_TILE
        cptc = jnp.where(cid == 0, cpt0, cpt1)
        pltpu.sync_copy(zeros_hbm.at[pl.ds(r0, ROWS_PER_TILE)],
                        acc_sh.at[pl.ds(r0, ROWS_PER_TILE)])
        pltpu.sync_copy(src_hbm.at[wid], src_v)
        pltpu.sync_copy(dst_hbm.at[wid], dst_v)
        plsc.subcore_barrier()

        def step(j, carry):
            pltpu.async_copy(h_hbm.at[src_v.at[j]], rows_v, gsem).wait()
            pltpu.sync_copy(rows_v, acc_sh.at[dst_v.at[j]], add=True)
            return carry

        lax.fori_loop(0, cptc, step, 0)
        plsc.subcore_barrier()
        pltpu.sync_copy(acc_sh.at[pl.ds(r0, ROWS_PER_TILE)],
                        out_hbm.at[pl.ds(cid * ACC_ROWS + r0, ROWS_PER_TILE)])

    return agg


def _make_deg(ept):
    @functools.partial(
        pl.kernel,
        out_type=jax.ShapeDtypeStruct((NW, ACC_ROWS), jnp.float32),
        mesh=_mesh,
        scratch_types=[
            pltpu.VMEM((ept,), jnp.int32),
            pltpu.VMEM((ACC_ROWS,), jnp.float32),
        ],
        compiler_params=pltpu.CompilerParams(needs_layout_passes=False),
    )
    def deg(dst_hbm, out_hbm, dst_v, deg_v):
        cid = lax.axis_index("c")
        sid = lax.axis_index("s")
        wid = sid * NC + cid
        pltpu.sync_copy(dst_hbm.at[wid], dst_v)

        def zstep(i, carry):
            deg_v[pl.ds(pl.multiple_of(i * 16, 16), 16)] = jnp.zeros((16,), jnp.float32)
            return carry

        lax.fori_loop(0, ACC_ROWS // 16, zstep, 0)
        ones = jnp.ones((16,), jnp.float32)

        def estep(j, carry):
            idx = dst_v[pl.ds(pl.multiple_of(j * 16, 16), 16)]
            plsc.addupdate_scatter(deg_v, [idx], ones)
            return carry

        lax.fori_loop(0, ept // 16, estep, 0)
        pltpu.sync_copy(deg_v, out_hbm.at[wid])

    return deg


def _dense_body(relu, agg_a, agg_b, degt, h, wn, ws, b, o_ref):
    deg = jnp.maximum(jnp.sum(degt[...], axis=1, keepdims=True), 1.0)
    agg = (agg_a[...] + agg_b[...]) / deg
    y = (jnp.dot(agg, wn[...], preferred_element_type=jnp.float32)
         + jnp.dot(h[...], ws[...], preferred_element_type=jnp.float32)
         + b[...])
    o_ref[...] = jnp.maximum(y, 0.0) if relu else y


def _make_dense(relu):
    bn = 1000
    grid = N_NODES // bn
    return pl.pallas_call(
        functools.partial(_dense_body, relu),
        grid=(grid,),
        in_specs=[
            pl.BlockSpec((bn, FEAT), lambda i: (i, 0)),
            pl.BlockSpec((bn, FEAT), lambda i: (i, 0)),
            pl.BlockSpec((bn, NW), lambda i: (i, 0)),
            pl.BlockSpec((bn, FEAT), lambda i: (i, 0)),
            pl.BlockSpec((FEAT, FEAT), lambda i: (0, 0)),
            pl.BlockSpec((FEAT, FEAT), lambda i: (0, 0)),
            pl.BlockSpec((1, FEAT), lambda i: (0, 0)),
        ],
        out_specs=pl.BlockSpec((bn, FEAT), lambda i: (i, 0)),
        out_shape=jax.ShapeDtypeStruct((N_NODES, FEAT), jnp.float32),
    )


def _pool_body(agg_a, agg_b, degt, h2, wn, ws, b, gw, gb, rw, rb, bids, o_ref):
    deg = jnp.maximum(jnp.sum(degt[...], axis=1, keepdims=True), 1.0)
    agg = (agg_a[...] + agg_b[...]) / deg
    h3 = (jnp.dot(agg, wn[...], preferred_element_type=jnp.float32)
          + jnp.dot(h2[...], ws[...], preferred_element_type=jnp.float32)
          + b[...])
    gate = jnp.dot(h3, gw[...], preferred_element_type=jnp.float32) + gb[...]
    gids = lax.broadcasted_iota(jnp.int32, (N_NODES, G), 1)
    mask = bids[...] == gids
    maskf = mask.astype(jnp.float32)
    gmax = jnp.max(jnp.where(mask, gate, -1e30), axis=0, keepdims=True)
    gmax_n = jnp.sum(jnp.where(mask, gmax, 0.0), axis=1, keepdims=True)
    e = jnp.exp(gate - gmax_n)
    denom = jnp.sum(e * maskf, axis=0, keepdims=True)
    denom_n = jnp.sum(jnp.where(mask, denom, 0.0), axis=1, keepdims=True)
    wgt = e / denom_n
    wm = maskf * wgt
    pooled = lax.dot_general(wm, h3, (((0,), (0,)), ((), ())),
                             preferred_element_type=jnp.float32)
    o_ref[...] = jnp.tanh(
        jnp.dot(pooled, rw[...], preferred_element_type=jnp.float32) + rb[...])


_pool = pl.pallas_call(
    _pool_body,
    out_shape=jax.ShapeDtypeStruct((G, OUT), jnp.float32),
)


def kernel(x, edge_index, batch, Wn1, Ws1, b1, Wn2, Ws2, b2, Wn3, Ws3, b3,
           gate_W, gate_b, reg_W, reg_b):
    e = edge_index.shape[1]
    src = edge_index[0]
    dst = edge_index[1]

    cpt_sum = -(-e // (NS * CHUNK))
    cpt0 = max(1, min(cpt_sum - 1, round(SPLIT0 * cpt_sum)))
    cpt1 = cpt_sum - cpt0
    pad = NS * cpt_sum * CHUNK - e

    def blocks(a, padval):
        ap = jnp.concatenate([a, jnp.full((pad,), padval, a.dtype)])
        c0 = ap[:NS * cpt0 * CHUNK].reshape(NS, 1, cpt0, CHUNK)
        c1 = ap[NS * cpt0 * CHUNK:].reshape(NS, 1, cpt1, CHUNK)
        cptm = max(cpt0, cpt1)
        c0 = jnp.pad(c0, ((0, 0), (0, 0), (0, cptm - cpt0), (0, 0)))
        c1 = jnp.pad(c1, ((0, 0), (0, 0), (0, cptm - cpt1), (0, 0)))
        return jnp.concatenate([c0, c1], axis=1).reshape(NW, cptm, CHUNK)

    src_p = blocks(src, 0)
    dst_p = blocks(dst, N_NODES)

    ept_deg = -(-e // NW // 16) * 16
    padd = NW * ept_deg - e
    dst_flat = jnp.concatenate(
        [dst, jnp.full((padd,), N_NODES, jnp.int32)]).reshape(NW, ept_deg) if padd \
        else dst.reshape(NW, ept_deg)
    zeros = jnp.zeros((ACC_ROWS, FEAT), jnp.float32)

    agg_call = _make_agg((cpt0, cpt1))
    deg_call = _make_deg(ept_deg)

    degp = deg_call(dst_flat)
    degt = degp.T[:N_NODES]

    def split(slab):
        a = lax.slice(slab, (0, 0), (N_NODES, FEAT))
        b = lax.slice(slab, (ACC_ROWS, 0), (ACC_ROWS + N_NODES, FEAT))
        return a, b

    b1r = b1.reshape(1, FEAT)
    b2r = b2.reshape(1, FEAT)
    b3r = b3.reshape(1, FEAT)

    a1a, a1b = split(agg_call(x, src_p, dst_p, zeros))
    h1 = _make_dense(True)(a1a, a1b, degt, x, Wn1, Ws1, b1r)
    a2a, a2b = split(agg_call(h1, src_p, dst_p, zeros))
    h2 = _make_dense(True)(a2a, a2b, degt, h1, Wn2, Ws2, b2r)
    a3a, a3b = split(agg_call(h2, src_p, dst_p, zeros))
    out = _pool(a3a, a3b, degt, h2, Wn3, Ws3, b3r,
                gate_W, gate_b.reshape(1, 1), reg_W, reg_b.reshape(1, OUT),
                batch.reshape(N_NODES, 1))
    return out
